# baseline (speedup 1.0000x reference)
import functools

import jax
import jax.numpy as jnp
from jax import lax
from jax.experimental import pallas as pl
from jax.experimental.pallas import tpu as pltpu
from jax.experimental.pallas import tpu_sc as plsc

B = 8
N = 2048
F = 64

_NC = 2
_NS = 16
_NW = _NC * _NS

_INTERPRET = False


def _silu(x):
    return x * jax.nn.sigmoid(x)


def _dot(a, b):
    return jnp.dot(a.astype(jnp.bfloat16), b.astype(jnp.bfloat16),
                   preferred_element_type=jnp.float32)


def _geo_body(n_in, n_out, K, pos_ref, post_ref, idx_ref):
    bm = idx_ref.shape[1]
    c = pos_ref[0]
    cx = c[:, 0:1]
    cy = c[:, 1:2]
    cz = c[:, 2:3]
    px = post_ref[0, 0:1, :]
    py = post_ref[0, 1:2, :]
    pz = post_ref[0, 2:3, :]
    dx = cx - px
    dy = cy - py
    dz = cz - pz
    d = dx * dx + dy * dy + dz * dz
    col = lax.broadcasted_iota(jnp.int32, (bm, n_in), 1)
    cols = []
    for k in range(K):
        mn = jnp.min(d, axis=1, keepdims=True)
        am = jnp.min(jnp.where(d == mn, col, n_in), axis=1, keepdims=True)
        cols.append(am)
        if k < K - 1:
            d = jnp.where(col == am, jnp.inf, d)
    base = pl.program_id(0) * n_in
    idx_ref[0] = jnp.concatenate(cols, axis=1) + base


def _geo(pos, pos_t, n_in, n_out, K, bm):
    grid = (B, n_out // bm)
    return pl.pallas_call(
        functools.partial(_geo_body, n_in, n_out, K),
        grid=grid,
        in_specs=[
            pl.BlockSpec((1, bm, 3), lambda b, r: (b, r, 0)),
            pl.BlockSpec((1, 3, n_in), lambda b, r: (b, 0, 0)),
        ],
        out_specs=pl.BlockSpec((1, bm, K), lambda b, r: (b, r, 0)),
        out_shape=jax.ShapeDtypeStruct((B, n_out, K), jnp.int32),
        interpret=_INTERPRET,
    )(pos[:, :n_out], pos_t[:, :, :n_in])


def _ln(x, g, b):
    m = jnp.mean(x, axis=-1, keepdims=True)
    v = jnp.mean((x - m) ** 2, axis=-1, keepdims=True)
    return (x - m) / jnp.sqrt(v + 1e-5) * g + b


def _mlp_stack(h, Wls, bls):
    for i in range(Wls.shape[0]):
        h = jnp.maximum(_dot(h, Wls[i]) + bls[i], 0.0)
    return h


def _feat(G, crep, fdim):
    rel = G[:, fdim:] - crep
    return jnp.concatenate([G[:, :fdim], rel], axis=1)


def _dense0_body(g_ref, crep_ref, pos_ref, ch_ref, W_ref, b_ref, Wp_ref,
                 bp_ref, Wr_ref, g0_ref, be0_ref, tbl_ref):
    feat = _feat(g_ref[0, :, :68], crep_ref[0], 64)
    pre = _dot(feat, W_ref[...]) + b_ref[...]
    h = _silu(pre).reshape(1024, 3, 32)
    h = jnp.max(h, axis=1)
    h = _mlp_stack(h, Wp_ref[...], bp_ref[...])
    res = _dot(ch_ref[0], Wr_ref[...])
    ch1 = _ln(h, g0_ref[...], be0_ref[...]) + res
    tbl_ref[0, :, 0:32] = ch1
    tbl_ref[0, :, 32:35] = pos_ref[0]
    tbl_ref[0, :, 35:128] = jnp.zeros((1024, 93), jnp.float32)


def _dense1_body(g_ref, crep_ref, pos_ref, t1_ref, W_ref, b_ref, Wp_ref,
                 bp_ref, Wr_ref, g1_ref, be1_ref, tbl_ref, res2_ref):
    feat = _feat(g_ref[0, :, :36], crep_ref[0], 32)
    pre = _dot(feat, W_ref[...]) + b_ref[...]
    h = _silu(pre).reshape(512, 2, 128)
    h = jnp.max(h, axis=1)
    h = _mlp_stack(h, Wp_ref[...], bp_ref[...])
    ch2 = _silu(_ln(h, g1_ref[...], be1_ref[...]))
    tbl_ref[0] = ch2
    res2_ref[0] = _dot(t1_ref[0, :256, 0:32], Wr_ref[...])


def _dense2_body(gch_ref, gpos_ref, crep_ref, res2_ref, W_ref, b_ref, Wp_ref,
                 bp_ref, g2_ref, be2_ref, L1_ref, lb1_ref, L2_ref, lb2_ref,
                 L3_ref, lb3_ref, out_ref):
    rel = gpos_ref[0, :, 0:4] - crep_ref[0]
    feat = jnp.concatenate([gch_ref[0], rel], axis=1)
    pre = _dot(feat, W_ref[...]) + b_ref[...]
    h = _silu(pre).reshape(256, 2, 128)
    h = jnp.max(h, axis=1)
    h = _mlp_stack(h, Wp_ref[...], bp_ref[...])
    ch3 = _silu(_ln(h, g2_ref[...], be2_ref[...])) + res2_ref[0]
    h = _silu(_dot(ch3, L1_ref[...]) + lb1_ref[...])
    h = _silu(_dot(h, L2_ref[...]) + lb2_ref[...])
    out_ref[0] = _dot(h, L3_ref[...]) + lb3_ref[...]


def _full_spec(shape):
    n = len(shape)
    return pl.BlockSpec(shape, lambda b: (0,) * n)


def _batch_spec(shape):
    n = len(shape)
    return pl.BlockSpec((1,) + shape, lambda b: (b,) + (0,) * n)


def _dense0(G, crep, pos, ch, p):
    W = jnp.zeros((68, 32), jnp.float32).at[:67].set(p['W0'])
    return pl.pallas_call(
        _dense0_body,
        grid=(B,),
        in_specs=[
            _batch_spec((3072, 128)),
            _batch_spec((3072, 4)),
            _batch_spec((1024, 3)),
            _batch_spec((1024, 64)),
            _full_spec((68, 32)),
            _full_spec((32,)),
            _full_spec((10, 32, 32)),
            _full_spec((10, 32)),
            _full_spec((64, 32)),
            _full_spec((32,)),
            _full_spec((32,)),
        ],
        out_specs=_batch_spec((1024, 128)),
        out_shape=jax.ShapeDtypeStruct((B, 1024, 128), jnp.float32),
        interpret=_INTERPRET,
    )(G, crep, pos[:, :1024], ch[:, :1024], W, p['b0'], p['Wp0'], p['bp0'],
      p['Wr0'], p['g0'], p['be0'])


def _dense1(G, crep, pos, tbl1, p):
    W = jnp.zeros((36, 128), jnp.float32).at[:35].set(p['W1'])
    return pl.pallas_call(
        _dense1_body,
        grid=(B,),
        in_specs=[
            _batch_spec((1024, 128)),
            _batch_spec((1024, 4)),
            _batch_spec((512, 3)),
            _batch_spec((512, 36)),
            _full_spec((36, 128)),
            _full_spec((128,)),
            _full_spec((5, 128, 128)),
            _full_spec((5, 128)),
            _full_spec((32, 128)),
            _full_spec((128,)),
            _full_spec((128,)),
        ],
        out_specs=[
            _batch_spec((512, 128)),
            _batch_spec((256, 128)),
        ],
        out_shape=[
            jax.ShapeDtypeStruct((B, 512, 128), jnp.float32),
            jax.ShapeDtypeStruct((B, 256, 128), jnp.float32),
        ],
        interpret=_INTERPRET,
    )(G, crep, pos[:, :512], tbl1[:, :512], W, p['b1'], p['Wp1'], p['bp1'],
      p['Wr1'], p['g1'], p['be1'])


def _dense2(Gch, Gpos, crep, res2, p):
    W = jnp.zeros((132, 128), jnp.float32).at[:131].set(p['W2'])
    return pl.pallas_call(
        _dense2_body,
        grid=(B,),
        in_specs=[
            _batch_spec((512, 128)),
            _batch_spec((512, 128)),
            _batch_spec((512, 4)),
            _batch_spec((256, 128)),
            _full_spec((132, 128)),
            _full_spec((128,)),
            _full_spec((5, 128, 128)),
            _full_spec((5, 128)),
            _full_spec((128,)),
            _full_spec((128,)),
            _full_spec((128, 32)),
            _full_spec((32,)),
            _full_spec((32, 16)),
            _full_spec((16,)),
            _full_spec((16, 1)),
            _full_spec((1,)),
        ],
        out_specs=_batch_spec((256, 1)),
        out_shape=jax.ShapeDtypeStruct((B, 256, 1), jnp.float32),
        interpret=_INTERPRET,
    )(Gch, Gpos, crep, res2, W, p['b2'], p['Wp2'], p['bp2'],
      p['g2'], p['be2'], p['L1'], p['lb1'], p['L2'], p['lb2'],
      p['L3'], p['lb3'])


def _sc_geo12(posx, posy, posz):
    mesh = plsc.VectorSubcoreMesh(core_axis_name="c", subcore_axis_name="s")
    INF = jnp.float32(jnp.inf)

    def body(px_hbm, py_hbm, pz_hbm, o0_hbm, o1_hbm, o2_hbm,
             ptsx, ptsy, ptsz, ob0, ob1, ob2):
        w = lax.axis_index("s") * _NC + lax.axis_index("c")
        b = w // 4
        pltpu.sync_copy(px_hbm.at[pl.ds(b * 2048, 2048)], ptsx)
        pltpu.sync_copy(py_hbm.at[pl.ds(b * 2048, 2048)], ptsy)
        pltpu.sync_copy(pz_hbm.at[pl.ds(b * 2048, 2048)], ptsz)
        lane = lax.broadcasted_iota(jnp.int32, (16,), 0)

        i0 = 768 + (w % 4) * 64
        for g in range(4):
            cx = ptsx[pl.ds(i0 + 16 * g, 16)]
            cy = ptsy[pl.ds(i0 + 16 * g, 16)]
            cz = ptsz[pl.ds(i0 + 16 * g, 16)]

            def pt_chunk0(t, carry):
                m1, m2, m3, i1, i2, i3 = carry
                bx = ptsx[pl.ds(16 * t, 16)]
                by = ptsy[pl.ds(16 * t, 16)]
                bz = ptsz[pl.ds(16 * t, 16)]
                for jj in range(16):
                    dx = cx - bx[jj]
                    dy = cy - by[jj]
                    dz = cz - bz[jj]
                    d = dx * dx + dy * dy + dz * dz
                    jv = jnp.full((16,), 16 * t + jj, jnp.int32)
                    lt1 = d < m1
                    lt2 = d < m2
                    lt3 = d < m3
                    m3 = jnp.where(lt2, m2, jnp.where(lt3, d, m3))
                    i3 = jnp.where(lt2, i2, jnp.where(lt3, jv, i3))
                    m2 = jnp.where(lt1, m1, jnp.where(lt2, d, m2))
                    i2 = jnp.where(lt1, i1, jnp.where(lt2, jv, i2))
                    m1 = jnp.where(lt1, d, m1)
                    i1 = jnp.where(lt1, jv, i1)
                return m1, m2, m3, i1, i2, i3

            init = (jnp.full((16,), INF), jnp.full((16,), INF),
                    jnp.full((16,), INF), jnp.zeros((16,), jnp.int32),
                    jnp.zeros((16,), jnp.int32), jnp.zeros((16,), jnp.int32))
            m1, m2, m3, i1, i2, i3 = lax.fori_loop(0, 128, pt_chunk0, init)
            base = b * 2048
            sidx = 3 * (16 * g + lane)
            plsc.store_scatter(ob0, [sidx], i1 + base)
            plsc.store_scatter(ob0, [sidx + 1], i2 + base)
            plsc.store_scatter(ob0, [sidx + 2], i3 + base)
        pltpu.sync_copy(ob0, o0_hbm.at[pl.ds(w * 192, 192)])

        def run_layer(n_in, ncw, ob, o_hbm):
            i0 = (w % 4) * ncw
            for g in range(ncw // 16):
                cx = ptsx[pl.ds(i0 + 16 * g, 16)]
                cy = ptsy[pl.ds(i0 + 16 * g, 16)]
                cz = ptsz[pl.ds(i0 + 16 * g, 16)]

                def pt_chunk(t, carry):
                    m1, m2, i1, i2 = carry
                    bx = ptsx[pl.ds(16 * t, 16)]
                    by = ptsy[pl.ds(16 * t, 16)]
                    bz = ptsz[pl.ds(16 * t, 16)]
                    for jj in range(16):
                        dx = cx - bx[jj]
                        dy = cy - by[jj]
                        dz = cz - bz[jj]
                        d = dx * dx + dy * dy + dz * dz
                        jv = jnp.full((16,), 16 * t + jj, jnp.int32)
                        lt1 = d < m1
                        lt2 = d < m2
                        m2n = jnp.where(lt2, d, m2)
                        i2n = jnp.where(lt2, jv, i2)
                        m2 = jnp.where(lt1, m1, m2n)
                        i2 = jnp.where(lt1, i1, i2n)
                        m1 = jnp.where(lt1, d, m1)
                        i1 = jnp.where(lt1, jv, i1)
                    return m1, m2, i1, i2

                init = (jnp.full((16,), INF), jnp.full((16,), INF),
                        jnp.zeros((16,), jnp.int32), jnp.zeros((16,), jnp.int32))
                m1, m2, i1, i2 = lax.fori_loop(0, n_in // 16, pt_chunk, init)
                base = b * n_in
                sidx = 2 * (16 * g + lane)
                plsc.store_scatter(ob, [sidx], i1 + base)
                plsc.store_scatter(ob, [sidx + 1], i2 + base)
            pltpu.sync_copy(ob, o_hbm.at[pl.ds(w * 2 * ncw, 2 * ncw)])

        run_layer(1024, 128, ob1, o1_hbm)
        run_layer(512, 64, ob2, o2_hbm)

    f = pl.kernel(
        body,
        out_type=[jax.ShapeDtypeStruct((6144,), jnp.int32),
                  jax.ShapeDtypeStruct((8192,), jnp.int32),
                  jax.ShapeDtypeStruct((4096,), jnp.int32)],
        mesh=mesh,
        scratch_types=[
            pltpu.VMEM((2048,), jnp.float32),
            pltpu.VMEM((2048,), jnp.float32),
            pltpu.VMEM((2048,), jnp.float32),
            pltpu.VMEM((192,), jnp.int32),
            pltpu.VMEM((256,), jnp.int32),
            pltpu.VMEM((128,), jnp.int32),
        ],
        compiler_params=pltpu.CompilerParams(use_tc_tiling_on_sc=False,
                                             needs_layout_passes=False),
    )
    return f(posx, posy, posz)


def _sc_gather(tbls, gidx_flat, n_chunks):
    chunk = n_chunks * 128
    T = _NW * chunk
    M = len(tbls)
    mesh = plsc.VectorSubcoreMesh(core_axis_name="c", subcore_axis_name="s")

    def body(*refs):
        tbl_hbm = refs[:M]
        idx_hbm = refs[M]
        out_hbm = refs[M + 1:M + 1 + M]
        idx_v = refs[M + 1 + M]
        rows_v = refs[M + 2 + M:M + 2 + 2 * M]
        sem = refs[-1]
        wid = lax.axis_index("s") * _NC + lax.axis_index("c")
        base = wid * chunk
        pltpu.sync_copy(idx_hbm.at[pl.ds(base, chunk)], idx_v)
        copies = [
            pltpu.async_copy(tbl_hbm[m].at[idx_v.at[pl.ds(j * 128, 128)]],
                             rows_v[m].at[pl.ds(j * 128, 128)], sem)
            for m in range(M)
            for j in range(n_chunks)
        ]
        for c in copies:
            c.wait()
        for m in range(M):
            pltpu.sync_copy(rows_v[m], out_hbm[m].at[pl.ds(base, chunk)])

    f = pl.kernel(
        body,
        out_type=[jax.ShapeDtypeStruct((T, 128), jnp.float32)
                  for _ in range(M)],
        mesh=mesh,
        scratch_types=(
            [pltpu.VMEM((chunk,), jnp.int32)]
            + [pltpu.VMEM((chunk, 128), jnp.float32) for _ in range(M)]
            + [pltpu.SemaphoreType.DMA]
        ),
        compiler_params=pltpu.CompilerParams(use_tc_tiling_on_sc=False),
    )
    return f(*tbls, gidx_flat)


def kernel(position_matrix, channel_matrix, params, belonging):
    pos = position_matrix
    ch = channel_matrix
    pos_t = pos.transpose(0, 2, 1)

    idx0_tc = _geo(pos, pos_t, 2048, 768, 3, 256)
    idx0f_sc, idx1f, idx2f = _sc_geo12(pos_t[:, 0].reshape(-1),
                                       pos_t[:, 1].reshape(-1),
                                       pos_t[:, 2].reshape(-1))
    idx0 = jnp.concatenate([idx0_tc, idx0f_sc.reshape(B, 256, 3)], axis=1)

    posp = jnp.concatenate([pos, jnp.zeros((B, N, 1), jnp.float32)], axis=-1)
    tbl0 = jnp.concatenate(
        [ch, posp, jnp.zeros((B, N, 60), jnp.float32)], axis=-1)
    pos2tbl = jnp.concatenate(
        [posp[:, :512], jnp.zeros((B, 512, 124), jnp.float32)], axis=-1)
    crep0 = jnp.repeat(posp[:, :1024], 3, axis=1)
    crep1 = jnp.repeat(posp[:, :512], 2, axis=1)
    crep2 = jnp.repeat(posp[:, :256], 2, axis=1)

    (G0,) = _sc_gather([tbl0.reshape(B * 2048, 128)], idx0.reshape(-1), 6)
    tbl1 = _dense0(G0.reshape(B, 3072, 128), crep0, pos, ch, params)
    (G1,) = _sc_gather([tbl1.reshape(B * 1024, 128)], idx1f, 2)
    tbl2, res2 = _dense1(G1.reshape(B, 1024, 128), crep1, pos,
                         tbl1[:, :512, :36], params)
    G2ch, G2pos = _sc_gather(
        [tbl2.reshape(B * 512, 128), pos2tbl.reshape(B * 512, 128)],
        idx2f, 1)
    out = _dense2(G2ch.reshape(B, 512, 128), G2pos.reshape(B, 512, 128),
                  crep2, res2, params)
    return out


# speedup vs baseline: 1.0122x; 1.0122x over previous
import functools

import jax
import jax.numpy as jnp
from jax import lax
from jax.experimental import pallas as pl
from jax.experimental.pallas import tpu as pltpu
from jax.experimental.pallas import tpu_sc as plsc

B = 8
N = 2048
F = 64

_NC = 2
_NS = 16
_NW = _NC * _NS

_INTERPRET = False


def _silu(x):
    return x * jax.nn.sigmoid(x)


def _dot(a, b):
    return jnp.dot(a.astype(jnp.bfloat16), b.astype(jnp.bfloat16),
                   preferred_element_type=jnp.float32)


def _geo_body(n_in, n_out, K, pos_ref, post_ref, ch_ref, post2_ref,
              idx_ref, tbl_ref):
    tbl_ref[0, :, 0:64] = ch_ref[0]
    tbl_ref[0, :, 64:67] = post2_ref[0]
    tbl_ref[0, :, 67:128] = jnp.zeros((512, 61), jnp.float32)
    bm = idx_ref.shape[1]
    c = pos_ref[0]
    cx = c[:, 0:1]
    cy = c[:, 1:2]
    cz = c[:, 2:3]
    px = post_ref[0, 0:1, :]
    py = post_ref[0, 1:2, :]
    pz = post_ref[0, 2:3, :]
    dx = cx - px
    dy = cy - py
    dz = cz - pz
    d = dx * dx + dy * dy + dz * dz
    col = lax.broadcasted_iota(jnp.int32, (bm, n_in), 1)
    cols = []
    for k in range(K):
        mn = jnp.min(d, axis=1, keepdims=True)
        am = jnp.min(jnp.where(d == mn, col, n_in), axis=1, keepdims=True)
        cols.append(am)
        if k < K - 1:
            d = jnp.where(col == am, jnp.inf, d)
    base = pl.program_id(0) * n_in
    idx_ref[0] = jnp.concatenate(cols, axis=1) + base


def _geo(pos, pos_t, ch, n_in, n_out, K, bm):
    grid = (B, n_out // bm)
    return pl.pallas_call(
        functools.partial(_geo_body, n_in, n_out, K),
        grid=grid,
        in_specs=[
            pl.BlockSpec((1, bm, 3), lambda b, r: (b, r, 0)),
            pl.BlockSpec((1, 3, n_in), lambda b, r: (b, 0, 0)),
            pl.BlockSpec((1, 512, 64), lambda b, r: (b, r, 0)),
            pl.BlockSpec((1, 512, 3), lambda b, r: (b, r, 0)),
        ],
        out_specs=[
            pl.BlockSpec((1, bm, K), lambda b, r: (b, r, 0)),
            pl.BlockSpec((1, 512, 128), lambda b, r: (b, r, 0)),
        ],
        out_shape=[
            jax.ShapeDtypeStruct((B, n_out, K), jnp.int32),
            jax.ShapeDtypeStruct((B, 2048, 128), jnp.float32),
        ],
        interpret=_INTERPRET,
    )(pos[:, :n_out], pos_t[:, :, :n_in], ch, pos)


def _ln(x, g, b):
    m = jnp.mean(x, axis=-1, keepdims=True)
    v = jnp.mean((x - m) ** 2, axis=-1, keepdims=True)
    return (x - m) / jnp.sqrt(v + 1e-5) * g + b


def _mlp_stack(h, Wls, bls):
    for i in range(Wls.shape[0]):
        h = jnp.maximum(_dot(h, Wls[i]) + bls[i], 0.0)
    return h


def _feat(G, crep, fdim):
    rel = G[:, fdim:] - crep
    return jnp.concatenate([G[:, :fdim], rel], axis=1)


def _dense0_body(g_ref, crep_ref, pos_ref, ch_ref, W_ref, b_ref, Wp_ref,
                 bp_ref, Wr_ref, g0_ref, be0_ref, tbl_ref):
    feat = _feat(g_ref[0, :, :68], crep_ref[0], 64)
    pre = _dot(feat, W_ref[...]) + b_ref[...]
    h = _silu(pre).reshape(1024, 3, 32)
    h = jnp.max(h, axis=1)
    h = _mlp_stack(h, Wp_ref[...], bp_ref[...])
    res = _dot(ch_ref[0], Wr_ref[...])
    ch1 = _ln(h, g0_ref[...], be0_ref[...]) + res
    tbl_ref[0, :, 0:32] = ch1
    tbl_ref[0, :, 32:35] = pos_ref[0]
    tbl_ref[0, :, 35:128] = jnp.zeros((1024, 93), jnp.float32)


def _dense1_body(g_ref, crep_ref, pos_ref, t1_ref, W_ref, b_ref, Wp_ref,
                 bp_ref, Wr_ref, g1_ref, be1_ref, tbl_ref, res2_ref):
    feat = _feat(g_ref[0, :, :36], crep_ref[0], 32)
    pre = _dot(feat, W_ref[...]) + b_ref[...]
    h = _silu(pre).reshape(512, 2, 128)
    h = jnp.max(h, axis=1)
    h = _mlp_stack(h, Wp_ref[...], bp_ref[...])
    ch2 = _silu(_ln(h, g1_ref[...], be1_ref[...]))
    tbl_ref[0] = ch2
    res2_ref[0] = _dot(t1_ref[0, :256, 0:32], Wr_ref[...])


def _dense2_body(gch_ref, gpos_ref, crep_ref, res2_ref, W_ref, b_ref, Wp_ref,
                 bp_ref, g2_ref, be2_ref, L1_ref, lb1_ref, L2_ref, lb2_ref,
                 L3_ref, lb3_ref, out_ref):
    rel = gpos_ref[0, :, 0:4] - crep_ref[0]
    feat = jnp.concatenate([gch_ref[0], rel], axis=1)
    pre = _dot(feat, W_ref[...]) + b_ref[...]
    h = _silu(pre).reshape(256, 2, 128)
    h = jnp.max(h, axis=1)
    h = _mlp_stack(h, Wp_ref[...], bp_ref[...])
    ch3 = _silu(_ln(h, g2_ref[...], be2_ref[...])) + res2_ref[0]
    h = _silu(_dot(ch3, L1_ref[...]) + lb1_ref[...])
    h = _silu(_dot(h, L2_ref[...]) + lb2_ref[...])
    out_ref[0] = _dot(h, L3_ref[...]) + lb3_ref[...]


def _full_spec(shape):
    n = len(shape)
    return pl.BlockSpec(shape, lambda b: (0,) * n)


def _batch_spec(shape):
    n = len(shape)
    return pl.BlockSpec((1,) + shape, lambda b: (b,) + (0,) * n)


def _dense0(G, crep, pos, ch, p):
    W = jnp.zeros((68, 32), jnp.float32).at[:67].set(p['W0'])
    return pl.pallas_call(
        _dense0_body,
        grid=(B,),
        in_specs=[
            _batch_spec((3072, 128)),
            _batch_spec((3072, 4)),
            _batch_spec((1024, 3)),
            _batch_spec((1024, 64)),
            _full_spec((68, 32)),
            _full_spec((32,)),
            _full_spec((10, 32, 32)),
            _full_spec((10, 32)),
            _full_spec((64, 32)),
            _full_spec((32,)),
            _full_spec((32,)),
        ],
        out_specs=_batch_spec((1024, 128)),
        out_shape=jax.ShapeDtypeStruct((B, 1024, 128), jnp.float32),
        interpret=_INTERPRET,
    )(G, crep, pos[:, :1024], ch[:, :1024], W, p['b0'], p['Wp0'], p['bp0'],
      p['Wr0'], p['g0'], p['be0'])


def _dense1(G, crep, pos, tbl1, p):
    W = jnp.zeros((36, 128), jnp.float32).at[:35].set(p['W1'])
    return pl.pallas_call(
        _dense1_body,
        grid=(B,),
        in_specs=[
            _batch_spec((1024, 128)),
            _batch_spec((1024, 4)),
            _batch_spec((512, 3)),
            _batch_spec((512, 36)),
            _full_spec((36, 128)),
            _full_spec((128,)),
            _full_spec((5, 128, 128)),
            _full_spec((5, 128)),
            _full_spec((32, 128)),
            _full_spec((128,)),
            _full_spec((128,)),
        ],
        out_specs=[
            _batch_spec((512, 128)),
            _batch_spec((256, 128)),
        ],
        out_shape=[
            jax.ShapeDtypeStruct((B, 512, 128), jnp.float32),
            jax.ShapeDtypeStruct((B, 256, 128), jnp.float32),
        ],
        interpret=_INTERPRET,
    )(G, crep, pos[:, :512], tbl1[:, :512], W, p['b1'], p['Wp1'], p['bp1'],
      p['Wr1'], p['g1'], p['be1'])


def _dense2(Gch, Gpos, crep, res2, p):
    W = jnp.zeros((132, 128), jnp.float32).at[:131].set(p['W2'])
    return pl.pallas_call(
        _dense2_body,
        grid=(B,),
        in_specs=[
            _batch_spec((512, 128)),
            _batch_spec((512, 128)),
            _batch_spec((512, 4)),
            _batch_spec((256, 128)),
            _full_spec((132, 128)),
            _full_spec((128,)),
            _full_spec((5, 128, 128)),
            _full_spec((5, 128)),
            _full_spec((128,)),
            _full_spec((128,)),
            _full_spec((128, 32)),
            _full_spec((32,)),
            _full_spec((32, 16)),
            _full_spec((16,)),
            _full_spec((16, 1)),
            _full_spec((1,)),
        ],
        out_specs=_batch_spec((256, 1)),
        out_shape=jax.ShapeDtypeStruct((B, 256, 1), jnp.float32),
        interpret=_INTERPRET,
    )(Gch, Gpos, crep, res2, W, p['b2'], p['Wp2'], p['bp2'],
      p['g2'], p['be2'], p['L1'], p['lb1'], p['L2'], p['lb2'],
      p['L3'], p['lb3'])


def _sc_geo12(posx, posy, posz):
    mesh = plsc.VectorSubcoreMesh(core_axis_name="c", subcore_axis_name="s")
    INF = jnp.float32(jnp.inf)

    def body(px_hbm, py_hbm, pz_hbm, o1_hbm, o2_hbm,
             ptsx, ptsy, ptsz, ob1, ob2):
        w = lax.axis_index("s") * _NC + lax.axis_index("c")
        b = w // 4
        pltpu.sync_copy(px_hbm.at[pl.ds(b * 2048, 1024)], ptsx)
        pltpu.sync_copy(py_hbm.at[pl.ds(b * 2048, 1024)], ptsy)
        pltpu.sync_copy(pz_hbm.at[pl.ds(b * 2048, 1024)], ptsz)
        lane = lax.broadcasted_iota(jnp.int32, (16,), 0)

        def run_layer(n_in, ncw, ob, o_hbm):
            i0 = (w % 4) * ncw
            for g in range(ncw // 16):
                cx = ptsx[pl.ds(i0 + 16 * g, 16)]
                cy = ptsy[pl.ds(i0 + 16 * g, 16)]
                cz = ptsz[pl.ds(i0 + 16 * g, 16)]

                def pt_chunk(t, carry):
                    m1, m2, i1, i2 = carry
                    bx = ptsx[pl.ds(16 * t, 16)]
                    by = ptsy[pl.ds(16 * t, 16)]
                    bz = ptsz[pl.ds(16 * t, 16)]
                    for jj in range(16):
                        dx = cx - bx[jj]
                        dy = cy - by[jj]
                        dz = cz - bz[jj]
                        d = dx * dx + dy * dy + dz * dz
                        jv = jnp.full((16,), 16 * t + jj, jnp.int32)
                        lt1 = d < m1
                        lt2 = d < m2
                        m2n = jnp.where(lt2, d, m2)
                        i2n = jnp.where(lt2, jv, i2)
                        m2 = jnp.where(lt1, m1, m2n)
                        i2 = jnp.where(lt1, i1, i2n)
                        m1 = jnp.where(lt1, d, m1)
                        i1 = jnp.where(lt1, jv, i1)
                    return m1, m2, i1, i2

                init = (jnp.full((16,), INF), jnp.full((16,), INF),
                        jnp.zeros((16,), jnp.int32), jnp.zeros((16,), jnp.int32))
                m1, m2, i1, i2 = lax.fori_loop(0, n_in // 16, pt_chunk, init)
                base = b * n_in
                sidx = 2 * (16 * g + lane)
                plsc.store_scatter(ob, [sidx], i1 + base)
                plsc.store_scatter(ob, [sidx + 1], i2 + base)
            pltpu.sync_copy(ob, o_hbm.at[pl.ds(w * 2 * ncw, 2 * ncw)])

        run_layer(1024, 128, ob1, o1_hbm)
        run_layer(512, 64, ob2, o2_hbm)

    f = pl.kernel(
        body,
        out_type=[jax.ShapeDtypeStruct((8192,), jnp.int32),
                  jax.ShapeDtypeStruct((4096,), jnp.int32)],
        mesh=mesh,
        scratch_types=[
            pltpu.VMEM((1024,), jnp.float32),
            pltpu.VMEM((1024,), jnp.float32),
            pltpu.VMEM((1024,), jnp.float32),
            pltpu.VMEM((256,), jnp.int32),
            pltpu.VMEM((128,), jnp.int32),
        ],
        compiler_params=pltpu.CompilerParams(use_tc_tiling_on_sc=False,
                                             needs_layout_passes=False),
    )
    return f(posx, posy, posz)


def _sc_gather(tbls, gidx_flat, n_chunks):
    chunk = n_chunks * 128
    T = _NW * chunk
    M = len(tbls)
    mesh = plsc.VectorSubcoreMesh(core_axis_name="c", subcore_axis_name="s")

    def body(*refs):
        tbl_hbm = refs[:M]
        idx_hbm = refs[M]
        out_hbm = refs[M + 1:M + 1 + M]
        idx_v = refs[M + 1 + M]
        rows_v = refs[M + 2 + M:M + 2 + 2 * M]
        sem = refs[-1]
        wid = lax.axis_index("s") * _NC + lax.axis_index("c")
        base = wid * chunk
        pltpu.sync_copy(idx_hbm.at[pl.ds(base, chunk)], idx_v)
        copies = [
            pltpu.async_copy(tbl_hbm[m].at[idx_v.at[pl.ds(j * 128, 128)]],
                             rows_v[m].at[pl.ds(j * 128, 128)], sem)
            for m in range(M)
            for j in range(n_chunks)
        ]
        for c in copies:
            c.wait()
        for m in range(M):
            pltpu.sync_copy(rows_v[m], out_hbm[m].at[pl.ds(base, chunk)])

    f = pl.kernel(
        body,
        out_type=[jax.ShapeDtypeStruct((T, 128), jnp.float32)
                  for _ in range(M)],
        mesh=mesh,
        scratch_types=(
            [pltpu.VMEM((chunk,), jnp.int32)]
            + [pltpu.VMEM((chunk, 128), jnp.float32) for _ in range(M)]
            + [pltpu.SemaphoreType.DMA]
        ),
        compiler_params=pltpu.CompilerParams(use_tc_tiling_on_sc=False),
    )
    return f(*tbls, gidx_flat)


def kernel(position_matrix, channel_matrix, params, belonging):
    pos = position_matrix
    ch = channel_matrix
    pos_t = pos.transpose(0, 2, 1)

    idx0, tbl0 = _geo(pos, pos_t, ch, 2048, 1024, 3, 256)
    idx1f, idx2f = _sc_geo12(pos_t[:, 0].reshape(-1),
                             pos_t[:, 1].reshape(-1),
                             pos_t[:, 2].reshape(-1))

    posp = jnp.concatenate([pos, jnp.zeros((B, N, 1), jnp.float32)], axis=-1)
    pos2tbl = jnp.concatenate(
        [posp[:, :512], jnp.zeros((B, 512, 124), jnp.float32)], axis=-1)
    crep0 = jnp.repeat(posp[:, :1024], 3, axis=1)
    crep1 = jnp.repeat(posp[:, :512], 2, axis=1)
    crep2 = jnp.repeat(posp[:, :256], 2, axis=1)

    (G0,) = _sc_gather([tbl0.reshape(B * 2048, 128)], idx0.reshape(-1), 6)
    tbl1 = _dense0(G0.reshape(B, 3072, 128), crep0, pos, ch, params)
    (G1,) = _sc_gather([tbl1.reshape(B * 1024, 128)], idx1f, 2)
    tbl2, res2 = _dense1(G1.reshape(B, 1024, 128), crep1, pos,
                         tbl1[:, :512, :36], params)
    G2ch, G2pos = _sc_gather(
        [tbl2.reshape(B * 512, 128), pos2tbl.reshape(B * 512, 128)],
        idx2f, 1)
    out = _dense2(G2ch.reshape(B, 512, 128), G2pos.reshape(B, 512, 128),
                  crep2, res2, params)
    return out


# speedup vs baseline: 1.0495x; 1.0368x over previous
import functools

import jax
import jax.numpy as jnp
from jax import lax
from jax.experimental import pallas as pl
from jax.experimental.pallas import tpu as pltpu
from jax.experimental.pallas import tpu_sc as plsc

B = 8
N = 2048
F = 64

_NC = 2
_NS = 16
_NW = _NC * _NS

_INTERPRET = False


def _silu(x):
    return x * jax.nn.sigmoid(x)


def _dot(a, b):
    return jnp.dot(a.astype(jnp.bfloat16), b.astype(jnp.bfloat16),
                   preferred_element_type=jnp.float32)


def _geo_body(n_in, n_out, K, pos_ref, post_ref, idx_ref):
    bm = idx_ref.shape[1]
    c = pos_ref[0]
    cx = c[:, 0:1]
    cy = c[:, 1:2]
    cz = c[:, 2:3]
    px = post_ref[0, 0:1, :]
    py = post_ref[0, 1:2, :]
    pz = post_ref[0, 2:3, :]
    dx = cx - px
    dy = cy - py
    dz = cz - pz
    d = dx * dx + dy * dy + dz * dz
    col = lax.broadcasted_iota(jnp.int32, (bm, n_in), 1)
    cols = []
    for k in range(K):
        mn = jnp.min(d, axis=1, keepdims=True)
        am = jnp.min(jnp.where(d == mn, col, n_in), axis=1, keepdims=True)
        cols.append(am)
        if k < K - 1:
            d = jnp.where(col == am, jnp.inf, d)
    base = pl.program_id(0) * n_in
    idx_ref[0] = jnp.concatenate(cols, axis=1) + base


def _geo(pos, pos_t, n_in, n_out, K, bm):
    grid = (B, n_out // bm)
    return pl.pallas_call(
        functools.partial(_geo_body, n_in, n_out, K),
        grid=grid,
        in_specs=[
            pl.BlockSpec((1, bm, 3), lambda b, r: (b, r, 0)),
            pl.BlockSpec((1, 3, n_in), lambda b, r: (b, 0, 0)),
        ],
        out_specs=pl.BlockSpec((1, bm, K), lambda b, r: (b, r, 0)),
        out_shape=jax.ShapeDtypeStruct((B, n_out, K), jnp.int32),
        interpret=_INTERPRET,
    )(pos[:, :n_out], pos_t[:, :, :n_in])


def _ln(x, g, b):
    m = jnp.mean(x, axis=-1, keepdims=True)
    v = jnp.mean((x - m) ** 2, axis=-1, keepdims=True)
    return (x - m) / jnp.sqrt(v + 1e-5) * g + b


def _mlp_stack(h, Wls, bls):
    for i in range(Wls.shape[0]):
        h = jnp.maximum(_dot(h, Wls[i]) + bls[i], 0.0)
    return h


def _feat(G, crep, fdim):
    rel = G[:, fdim:] - crep
    return jnp.concatenate([G[:, :fdim], rel], axis=1)


def _dense0_body(g_ref, crep_ref, pos_ref, ch_ref, W_ref, b_ref, Wp_ref,
                 bp_ref, Wr_ref, g0_ref, be0_ref, tbl_ref):
    feat = _feat(g_ref[0, :, :68], crep_ref[0], 64)
    pre = _dot(feat, W_ref[...]) + b_ref[...]
    h = _silu(pre).reshape(1024, 3, 32)
    h = jnp.max(h, axis=1)
    h = _mlp_stack(h, Wp_ref[...], bp_ref[...])
    res = _dot(ch_ref[0], Wr_ref[...])
    ch1 = _ln(h, g0_ref[...], be0_ref[...]) + res
    tbl_ref[0, :, 0:32] = ch1
    tbl_ref[0, :, 32:35] = pos_ref[0]
    tbl_ref[0, :, 35:128] = jnp.zeros((1024, 93), jnp.float32)


def _dense1_body(g_ref, crep_ref, pos_ref, t1_ref, W_ref, b_ref, Wp_ref,
                 bp_ref, Wr_ref, g1_ref, be1_ref, tbl_ref, res2_ref):
    feat = _feat(g_ref[0, :, :36], crep_ref[0], 32)
    pre = _dot(feat, W_ref[...]) + b_ref[...]
    h = _silu(pre).reshape(512, 2, 128)
    h = jnp.max(h, axis=1)
    h = _mlp_stack(h, Wp_ref[...], bp_ref[...])
    ch2 = _silu(_ln(h, g1_ref[...], be1_ref[...]))
    tbl_ref[0] = ch2
    res2_ref[0] = _dot(t1_ref[0, :256, 0:32], Wr_ref[...])


def _dense2_body(gch_ref, gpos_ref, crep_ref, res2_ref, W_ref, b_ref, Wp_ref,
                 bp_ref, g2_ref, be2_ref, L1_ref, lb1_ref, L2_ref, lb2_ref,
                 L3_ref, lb3_ref, out_ref):
    rel = gpos_ref[0, :, 0:4] - crep_ref[0]
    feat = jnp.concatenate([gch_ref[0], rel], axis=1)
    pre = _dot(feat, W_ref[...]) + b_ref[...]
    h = _silu(pre).reshape(256, 2, 128)
    h = jnp.max(h, axis=1)
    h = _mlp_stack(h, Wp_ref[...], bp_ref[...])
    ch3 = _silu(_ln(h, g2_ref[...], be2_ref[...])) + res2_ref[0]
    h = _silu(_dot(ch3, L1_ref[...]) + lb1_ref[...])
    h = _silu(_dot(h, L2_ref[...]) + lb2_ref[...])
    out_ref[0] = _dot(h, L3_ref[...]) + lb3_ref[...]


def _full_spec(shape):
    n = len(shape)
    return pl.BlockSpec(shape, lambda b: (0,) * n)


def _batch_spec(shape):
    n = len(shape)
    return pl.BlockSpec((1,) + shape, lambda b: (b,) + (0,) * n)


def _dense0(G, crep, pos, ch, p):
    W = jnp.zeros((68, 32), jnp.float32).at[:67].set(p['W0'])
    return pl.pallas_call(
        _dense0_body,
        grid=(B,),
        in_specs=[
            _batch_spec((3072, 128)),
            _batch_spec((3072, 4)),
            _batch_spec((1024, 3)),
            _batch_spec((1024, 64)),
            _full_spec((68, 32)),
            _full_spec((32,)),
            _full_spec((10, 32, 32)),
            _full_spec((10, 32)),
            _full_spec((64, 32)),
            _full_spec((32,)),
            _full_spec((32,)),
        ],
        out_specs=_batch_spec((1024, 128)),
        out_shape=jax.ShapeDtypeStruct((B, 1024, 128), jnp.float32),
        interpret=_INTERPRET,
    )(G, crep, pos[:, :1024], ch[:, :1024], W, p['b0'], p['Wp0'], p['bp0'],
      p['Wr0'], p['g0'], p['be0'])


def _dense1(G, crep, pos, tbl1, p):
    W = jnp.zeros((36, 128), jnp.float32).at[:35].set(p['W1'])
    return pl.pallas_call(
        _dense1_body,
        grid=(B,),
        in_specs=[
            _batch_spec((1024, 128)),
            _batch_spec((1024, 4)),
            _batch_spec((512, 3)),
            _batch_spec((512, 36)),
            _full_spec((36, 128)),
            _full_spec((128,)),
            _full_spec((5, 128, 128)),
            _full_spec((5, 128)),
            _full_spec((32, 128)),
            _full_spec((128,)),
            _full_spec((128,)),
        ],
        out_specs=[
            _batch_spec((512, 128)),
            _batch_spec((256, 128)),
        ],
        out_shape=[
            jax.ShapeDtypeStruct((B, 512, 128), jnp.float32),
            jax.ShapeDtypeStruct((B, 256, 128), jnp.float32),
        ],
        interpret=_INTERPRET,
    )(G, crep, pos[:, :512], tbl1[:, :512], W, p['b1'], p['Wp1'], p['bp1'],
      p['Wr1'], p['g1'], p['be1'])


def _dense2(Gch, Gpos, crep, res2, p):
    W = jnp.zeros((132, 128), jnp.float32).at[:131].set(p['W2'])
    return pl.pallas_call(
        _dense2_body,
        grid=(B,),
        in_specs=[
            _batch_spec((512, 128)),
            _batch_spec((512, 128)),
            _batch_spec((512, 4)),
            _batch_spec((256, 128)),
            _full_spec((132, 128)),
            _full_spec((128,)),
            _full_spec((5, 128, 128)),
            _full_spec((5, 128)),
            _full_spec((128,)),
            _full_spec((128,)),
            _full_spec((128, 32)),
            _full_spec((32,)),
            _full_spec((32, 16)),
            _full_spec((16,)),
            _full_spec((16, 1)),
            _full_spec((1,)),
        ],
        out_specs=_batch_spec((256, 1)),
        out_shape=jax.ShapeDtypeStruct((B, 256, 1), jnp.float32),
        interpret=_INTERPRET,
    )(Gch, Gpos, crep, res2, W, p['b2'], p['Wp2'], p['bp2'],
      p['g2'], p['be2'], p['L1'], p['lb1'], p['L2'], p['lb2'],
      p['L3'], p['lb3'])


def _sc_geo12(posx, posy, posz):
    mesh = plsc.VectorSubcoreMesh(core_axis_name="c", subcore_axis_name="s")
    INF = jnp.float32(jnp.inf)

    def body(px_hbm, py_hbm, pz_hbm, o1_hbm, o2_hbm,
             ptsx, ptsy, ptsz, ob1, ob2):
        w = lax.axis_index("s") * _NC + lax.axis_index("c")
        b = w // 4
        pltpu.sync_copy(px_hbm.at[pl.ds(b * 2048, 1024)], ptsx)
        pltpu.sync_copy(py_hbm.at[pl.ds(b * 2048, 1024)], ptsy)
        pltpu.sync_copy(pz_hbm.at[pl.ds(b * 2048, 1024)], ptsz)
        lane = lax.broadcasted_iota(jnp.int32, (16,), 0)

        def run_layer(n_in, ncw, ob, o_hbm):
            i0 = (w % 4) * ncw
            for g in range(ncw // 16):
                cx = ptsx[pl.ds(i0 + 16 * g, 16)]
                cy = ptsy[pl.ds(i0 + 16 * g, 16)]
                cz = ptsz[pl.ds(i0 + 16 * g, 16)]

                def pt_chunk(t, carry):
                    m1, m2, i1, i2 = carry
                    bx = ptsx[pl.ds(16 * t, 16)]
                    by = ptsy[pl.ds(16 * t, 16)]
                    bz = ptsz[pl.ds(16 * t, 16)]
                    for jj in range(16):
                        dx = cx - bx[jj]
                        dy = cy - by[jj]
                        dz = cz - bz[jj]
                        d = dx * dx + dy * dy + dz * dz
                        jv = jnp.full((16,), 16 * t + jj, jnp.int32)
                        lt1 = d < m1
                        lt2 = d < m2
                        m2n = jnp.where(lt2, d, m2)
                        i2n = jnp.where(lt2, jv, i2)
                        m2 = jnp.where(lt1, m1, m2n)
                        i2 = jnp.where(lt1, i1, i2n)
                        m1 = jnp.where(lt1, d, m1)
                        i1 = jnp.where(lt1, jv, i1)
                    return m1, m2, i1, i2

                init = (jnp.full((16,), INF), jnp.full((16,), INF),
                        jnp.zeros((16,), jnp.int32), jnp.zeros((16,), jnp.int32))
                m1, m2, i1, i2 = lax.fori_loop(0, n_in // 16, pt_chunk, init)
                base = b * n_in
                sidx = 2 * (16 * g + lane)
                plsc.store_scatter(ob, [sidx], i1 + base)
                plsc.store_scatter(ob, [sidx + 1], i2 + base)
            pltpu.sync_copy(ob, o_hbm.at[pl.ds(w * 2 * ncw, 2 * ncw)])

        run_layer(1024, 128, ob1, o1_hbm)
        run_layer(512, 64, ob2, o2_hbm)

    f = pl.kernel(
        body,
        out_type=[jax.ShapeDtypeStruct((8192,), jnp.int32),
                  jax.ShapeDtypeStruct((4096,), jnp.int32)],
        mesh=mesh,
        scratch_types=[
            pltpu.VMEM((1024,), jnp.float32),
            pltpu.VMEM((1024,), jnp.float32),
            pltpu.VMEM((1024,), jnp.float32),
            pltpu.VMEM((256,), jnp.int32),
            pltpu.VMEM((128,), jnp.int32),
        ],
        compiler_params=pltpu.CompilerParams(use_tc_tiling_on_sc=False,
                                             needs_layout_passes=False),
    )
    return f(posx, posy, posz)


def _sc_gather(tbls, gidx_flat, n_chunks):
    chunk = n_chunks * 128
    T = _NW * chunk
    M = len(tbls)
    mesh = plsc.VectorSubcoreMesh(core_axis_name="c", subcore_axis_name="s")

    def body(*refs):
        tbl_hbm = refs[:M]
        idx_hbm = refs[M]
        out_hbm = refs[M + 1:M + 1 + M]
        idx_v = refs[M + 1 + M]
        rows_v = refs[M + 2 + M:M + 2 + 2 * M]
        sem = refs[-1]
        wid = lax.axis_index("s") * _NC + lax.axis_index("c")
        base = wid * chunk
        pltpu.sync_copy(idx_hbm.at[pl.ds(base, chunk)], idx_v)
        copies = [
            pltpu.async_copy(tbl_hbm[m].at[idx_v.at[pl.ds(j * 128, 128)]],
                             rows_v[m].at[pl.ds(j * 128, 128)], sem)
            for m in range(M)
            for j in range(n_chunks)
        ]
        for c in copies:
            c.wait()
        for m in range(M):
            pltpu.sync_copy(rows_v[m], out_hbm[m].at[pl.ds(base, chunk)])

    f = pl.kernel(
        body,
        out_type=[jax.ShapeDtypeStruct((T, 128), jnp.float32)
                  for _ in range(M)],
        mesh=mesh,
        scratch_types=(
            [pltpu.VMEM((chunk,), jnp.int32)]
            + [pltpu.VMEM((chunk, 128), jnp.float32) for _ in range(M)]
            + [pltpu.SemaphoreType.DMA]
        ),
        compiler_params=pltpu.CompilerParams(use_tc_tiling_on_sc=False),
    )
    return f(*tbls, gidx_flat)


def kernel(position_matrix, channel_matrix, params, belonging):
    pos = position_matrix
    ch = channel_matrix
    pos_t = pos.transpose(0, 2, 1)

    idx0 = _geo(pos, pos_t, 2048, 1024, 3, 256)
    idx1f, idx2f = _sc_geo12(pos_t[:, 0].reshape(-1),
                             pos_t[:, 1].reshape(-1),
                             pos_t[:, 2].reshape(-1))

    posp = jnp.concatenate([pos, jnp.zeros((B, N, 1), jnp.float32)], axis=-1)
    tbl0 = jnp.concatenate(
        [ch, posp, jnp.zeros((B, N, 60), jnp.float32)], axis=-1)
    pos2tbl = jnp.concatenate(
        [posp[:, :512], jnp.zeros((B, 512, 124), jnp.float32)], axis=-1)
    crep0 = jnp.repeat(posp[:, :1024], 3, axis=1)
    crep1 = jnp.repeat(posp[:, :512], 2, axis=1)
    crep2 = jnp.repeat(posp[:, :256], 2, axis=1)

    (G0,) = _sc_gather([tbl0.reshape(B * 2048, 128)], idx0.reshape(-1), 6)
    tbl1 = _dense0(G0.reshape(B, 3072, 128), crep0, pos, ch, params)
    (G1,) = _sc_gather([tbl1.reshape(B * 1024, 128)], idx1f, 2)
    tbl2, res2 = _dense1(G1.reshape(B, 1024, 128), crep1, pos,
                         tbl1[:, :512, :36], params)
    G2ch, G2pos = _sc_gather(
        [tbl2.reshape(B * 512, 128), pos2tbl.reshape(B * 512, 128)],
        idx2f, 1)
    out = _dense2(G2ch.reshape(B, 512, 128), G2pos.reshape(B, 512, 128),
                  crep2, res2, params)
    return out
